# baseline (speedup 1.0000x reference)
import functools

import jax
import jax.numpy as jnp
from jax import lax
from jax.experimental import pallas as pl
from jax.experimental.pallas import tpu as pltpu
from jax.experimental.pallas import tpu_sc as plsc

_R = 8


def _sc_kernel(B, S, D):
    info = plsc.get_sparse_core_info()
    NW = info.num_cores * info.num_subcores
    L = info.num_lanes
    s_per_w = S // NW
    nblk = s_per_w // _R
    ncol = D // L
    mesh = plsc.VectorSubcoreMesh(core_axis_name="c", subcore_axis_name="s")

    @functools.partial(
        pl.kernel,
        mesh=mesh,
        out_type=jax.ShapeDtypeStruct((B, S, D), jnp.float32),
        scratch_types=[
            pltpu.VMEM((_R, D), jnp.float32),
            pltpu.VMEM((B, _R, D), jnp.float32),
            pltpu.VMEM((B, _R, D), jnp.float32),
        ],
    )
    def k(x_hbm, pe_hbm, out_hbm, pe_v, x_v, o_v):
        wid = lax.axis_index("s") * info.num_cores + lax.axis_index("c")
        s_base = wid * s_per_w

        def blk(i, carry):
            s0 = s_base + i * _R
            pltpu.sync_copy(pe_hbm.at[pl.ds(s0, _R)], pe_v)
            for b in range(B):
                pltpu.sync_copy(x_hbm.at[b, pl.ds(s0, _R)], x_v.at[b])

            def row(r, carry2):
                def col(c, carry3):
                    d0 = c * L
                    pv = pe_v[r, pl.ds(d0, L)]
                    for b in range(B):
                        xv = x_v[b, r, pl.ds(d0, L)]
                        o_v[b, r, pl.ds(d0, L)] = jnp.where(xv == 0.0, 0.0, pv)
                    return carry3

                return lax.fori_loop(0, ncol, col, carry2)

            lax.fori_loop(0, _R, row, None)
            for b in range(B):
                pltpu.sync_copy(o_v.at[b], out_hbm.at[b, pl.ds(s0, _R)])
            return carry

        lax.fori_loop(0, nblk, blk, None)

    return k


def kernel(x, position_enc):
    B, S, D = x.shape
    pe = position_enc[:S]
    return _sc_kernel(B, S, D)(x, pe)
